# Initial kernel scaffold; baseline (speedup 1.0000x reference)
import jax
import jax.numpy as jnp
from jax.experimental import pallas as pl


def kernel(x, edge_index, W_l1, b_l1, W_r1, g1, be1, W_l2, b_l2, W_r2, g2, be2, Wc, bc):
    raise NotImplementedError("write your pallas kernel here")


# speedup vs baseline: 5.7805x; 5.7805x over previous
import functools

import jax
import jax.numpy as jnp
from jax import lax
from jax.experimental import pallas as pl
from jax.experimental.pallas import tpu as pltpu
from jax.experimental.pallas import tpu_sc as plsc

_NC = 2
_NS = 16
_C = 80
_CW = 16


def _sc_agg(tab, src1, dst1, with_cnt):
    n, d = tab.shape
    e = src1.shape[0]
    nw = _NC * _NS
    nchunks = e // _C // nw
    assert nchunks * _C * nw == e
    stripe = 640
    last_stripe = n - stripe * (_NS - 1)
    assert last_stripe > 0 and last_stripe % 8 == 0
    f32 = jnp.float32

    out_type = [jax.ShapeDtypeStruct((n, d), f32) for _ in range(_NC)]
    scratch = [
        pltpu.VMEM((_C,), jnp.int32),
        pltpu.VMEM((_C,), jnp.int32),
        pltpu.VMEM((_C, d), f32),
        pltpu.VMEM_SHARED((n, d), f32),
        pltpu.SemaphoreType.DMA,
    ]
    if with_cnt:
        out_type += [jax.ShapeDtypeStruct((n, _CW), f32) for _ in range(_NC)]
        scratch += [
            pltpu.VMEM((_C, _CW), f32),
            pltpu.VMEM_SHARED((n, _CW), f32),
        ]

    mesh = plsc.VectorSubcoreMesh(core_axis_name="c", subcore_axis_name="s")
    cparams = pltpu.CompilerParams(use_tc_tiling_on_sc=False)

    @functools.partial(pl.kernel, mesh=mesh, out_type=out_type,
                       scratch_types=scratch, compiler_params=cparams)
    def k(tab_h, src_h, dst_h, zd_h, zc_h, *rest):
        if with_cnt:
            (o0, o1, oc0, oc1, srcb, dstb, rows, acc, sem,
             ones_v, accc) = rest
        else:
            o0, o1, srcb, dstb, rows, acc, sem = rest
        cid = lax.axis_index("c")
        sid = lax.axis_index("s")
        wid = cid * _NS + sid
        r0 = sid * stripe

        def _each_stripe(fn):
            @pl.when(sid < _NS - 1)
            def _():
                fn(pl.ds(r0, stripe))

            @pl.when(sid == _NS - 1)
            def _():
                fn(pl.ds((_NS - 1) * stripe, last_stripe))

        def _init(rsl):
            pltpu.sync_copy(zd_h.at[rsl], acc.at[rsl])
            if with_cnt:
                pltpu.sync_copy(zc_h.at[rsl], accc.at[rsl])

        _each_stripe(_init)
        if with_cnt:
            for i in range(_C):
                ones_v[i] = jnp.ones((_CW,), f32)
        plsc.subcore_barrier()

        base = wid * (nchunks * _C)

        def step(j, carry):
            esl = pl.ds(base + j * _C, _C)
            pltpu.sync_copy(src_h.at[esl], srcb)
            pltpu.sync_copy(dst_h.at[esl], dstb)
            pltpu.async_copy(tab_h.at[srcb], rows, sem).wait()
            pltpu.sync_copy(rows, acc.at[dstb], add=True)
            if with_cnt:
                pltpu.sync_copy(ones_v, accc.at[dstb], add=True)
            return carry

        lax.fori_loop(0, nchunks, step, 0)
        plsc.subcore_barrier()

        def _writeout(rsl):
            @pl.when(cid == 0)
            def _():
                pltpu.sync_copy(acc.at[rsl], o0.at[rsl])
                if with_cnt:
                    pltpu.sync_copy(accc.at[rsl], oc0.at[rsl])

            @pl.when(cid == 1)
            def _():
                pltpu.sync_copy(acc.at[rsl], o1.at[rsl])
                if with_cnt:
                    pltpu.sync_copy(accc.at[rsl], oc1.at[rsl])

        _each_stripe(_writeout)

    zd = jnp.zeros((n, d), f32)
    zc = jnp.zeros((n, _CW), f32)
    return k(tab, src1, dst1, zd, zc)


def _dense1_body(a0, a1, c0, c1, xr, wl1, bl1, wr1, g1r, be1r, wl2, wr2,
                 oh2, ohr, ocnt):
    cnt = jnp.maximum((c0[...] + c1[...])[:, 0:1], 1.0)
    mean1 = (a0[...] + a1[...]) / cnt
    h = (jnp.dot(mean1, wl1[...], preferred_element_type=jnp.float32)
         + bl1[...]
         + jnp.dot(xr[...], wr1[...], preferred_element_type=jnp.float32))
    mu = jnp.mean(h, axis=0, keepdims=True)
    var = jnp.mean((h - mu) ** 2, axis=0, keepdims=True)
    h = (h - mu) / jnp.sqrt(var + 1e-5) * g1r[...] + be1r[...]
    h = jnp.maximum(h, 0.0)
    oh2[...] = jnp.dot(h, wl2[...], preferred_element_type=jnp.float32)
    ohr[...] = jnp.dot(h, wr2[...], preferred_element_type=jnp.float32)
    ocnt[...] = cnt


def _dense2_body(q0, q1, cnt_r, hr, bl2, g2r, be2r, wc, bc, out):
    h = (q0[...] + q1[...]) / cnt_r[...] + bl2[...] + hr[...]
    mu = jnp.mean(h, axis=0, keepdims=True)
    var = jnp.mean((h - mu) ** 2, axis=0, keepdims=True)
    h = (h - mu) / jnp.sqrt(var + 1e-5) * g2r[...] + be2r[...]
    h = jnp.maximum(h, 0.0)
    out[...] = jnp.dot(h, wc[...], preferred_element_type=jnp.float32) + bc[...]


def kernel(x, edge_index, W_l1, b_l1, W_r1, g1, be1, W_l2, b_l2, W_r2,
           g2, be2, Wc, bc):
    n, d = x.shape
    e = edge_index.shape[1]
    h2 = W_l2.shape[0]
    f32 = jnp.float32

    src1 = edge_index[0]
    dst1 = edge_index[1]

    a0, a1, c0, c1 = _sc_agg(x, src1, dst1, with_cnt=True)

    dense1 = pl.pallas_call(
        _dense1_body,
        out_shape=[
            jax.ShapeDtypeStruct((n, h2), f32),
            jax.ShapeDtypeStruct((n, h2), f32),
            jax.ShapeDtypeStruct((n, 1), f32),
        ],
    )
    h2pre, hr, cnt = dense1(
        a0, a1, c0, c1, x, W_l1.T, b_l1.reshape(1, -1), W_r1.T,
        g1.reshape(1, -1), be1.reshape(1, -1), W_l2.T, W_r2.T)

    q0, q1 = _sc_agg(h2pre, src1, dst1, with_cnt=False)

    wc_pad = jnp.zeros((h2, 8), f32).at[:, :2].set(Wc.T)
    bc_pad = jnp.zeros((1, 8), f32).at[0, :2].set(bc)
    dense2 = pl.pallas_call(
        _dense2_body,
        out_shape=jax.ShapeDtypeStruct((n, 8), f32),
    )
    logits8 = dense2(q0, q1, cnt, hr, b_l2.reshape(1, -1),
                     g2.reshape(1, -1), be2.reshape(1, -1), wc_pad, bc_pad)
    return logits8[:, :2]


# speedup vs baseline: 9.8784x; 1.7089x over previous
import functools

import jax
import jax.numpy as jnp
from jax import lax
from jax.experimental import pallas as pl
from jax.experimental.pallas import tpu as pltpu
from jax.experimental.pallas import tpu_sc as plsc

_NC = 2
_NS = 16
_C = 80
_KB = 5
_CW = 16


def _sc_agg(tab, src1, dst1, with_cnt):
    n, d = tab.shape
    e = src1.shape[0]
    nw = _NC * _NS
    nchunks = e // _C // nw
    nblk = nchunks // _KB
    assert nchunks * _C * nw == e and nblk * _KB == nchunks
    stripe = 640
    last_stripe = n - stripe * (_NS - 1)
    assert last_stripe > 0 and last_stripe % 8 == 0
    f32 = jnp.float32

    out_type = [jax.ShapeDtypeStruct((n, d), f32) for _ in range(_NC)]
    scratch = [
        pltpu.VMEM((_KB, _C), jnp.int32),
        pltpu.VMEM((_KB, _C), jnp.int32),
        pltpu.VMEM((_C, d), f32),
        pltpu.VMEM((_C, d), f32),
        pltpu.VMEM_SHARED((n, d), f32),
        pltpu.SemaphoreType.DMA,
        pltpu.SemaphoreType.DMA,
    ]
    if with_cnt:
        out_type += [jax.ShapeDtypeStruct((n, _CW), f32) for _ in range(_NC)]
        scratch += [
            pltpu.VMEM((_C, _CW), f32),
            pltpu.VMEM_SHARED((n, _CW), f32),
        ]

    mesh = plsc.VectorSubcoreMesh(core_axis_name="c", subcore_axis_name="s")
    cparams = pltpu.CompilerParams(use_tc_tiling_on_sc=False)

    @functools.partial(pl.kernel, mesh=mesh, out_type=out_type,
                       scratch_types=scratch, compiler_params=cparams)
    def k(tab_h, src_h, dst_h, zd_h, zc_h, *rest):
        if with_cnt:
            (o0, o1, oc0, oc1, srcb, dstb, rows0, rows1, acc, sem0, sem1,
             ones_v, accc) = rest
        else:
            o0, o1, srcb, dstb, rows0, rows1, acc, sem0, sem1 = rest
        rows = (rows0, rows1)
        sems = (sem0, sem1)
        cid = lax.axis_index("c")
        sid = lax.axis_index("s")
        wid = cid * _NS + sid
        r0 = sid * stripe

        def _each_stripe(fn):
            @pl.when(sid < _NS - 1)
            def _():
                fn(pl.ds(r0, stripe))

            @pl.when(sid == _NS - 1)
            def _():
                fn(pl.ds((_NS - 1) * stripe, last_stripe))

        def _init(rsl):
            pltpu.sync_copy(zd_h.at[rsl], acc.at[rsl])
            if with_cnt:
                pltpu.sync_copy(zc_h.at[rsl], accc.at[rsl])

        _each_stripe(_init)
        if with_cnt:
            for i in range(_C):
                ones_v[i] = jnp.ones((_CW,), f32)
        plsc.subcore_barrier()

        base = wid * nchunks

        def blk(g, carry):
            bsl = pl.ds(base + g * _KB, _KB)
            pltpu.sync_copy(src_h.at[bsl], srcb)
            pltpu.sync_copy(dst_h.at[bsl], dstb)
            handles = [None] * _KB

            def scat(j):
                handles[j].wait()
                pltpu.sync_copy(rows[j % 2], acc.at[dstb.at[j]], add=True)
                if with_cnt:
                    pltpu.sync_copy(ones_v, accc.at[dstb.at[j]], add=True)

            for j in range(_KB):
                handles[j] = pltpu.async_copy(
                    tab_h.at[srcb.at[j]], rows[j % 2], sems[j % 2])
                if j > 0:
                    scat(j - 1)
            scat(_KB - 1)
            return carry

        lax.fori_loop(0, nblk, blk, 0)
        plsc.subcore_barrier()

        def _writeout(rsl):
            @pl.when(cid == 0)
            def _():
                pltpu.sync_copy(acc.at[rsl], o0.at[rsl])
                if with_cnt:
                    pltpu.sync_copy(accc.at[rsl], oc0.at[rsl])

            @pl.when(cid == 1)
            def _():
                pltpu.sync_copy(acc.at[rsl], o1.at[rsl])
                if with_cnt:
                    pltpu.sync_copy(accc.at[rsl], oc1.at[rsl])

        _each_stripe(_writeout)

    zd = jnp.zeros((n, d), f32)
    zc = jnp.zeros((n, _CW), f32)
    return k(tab, src1.reshape(-1, _C), dst1.reshape(-1, _C), zd, zc)


def _dense1_body(a0, a1, c0, c1, xr, wl1, bl1, wr1, g1r, be1r, wl2, wr2,
                 oh2, ohr, ocnt):
    cnt = jnp.maximum((c0[...] + c1[...])[:, 0:1], 1.0)
    mean1 = (a0[...] + a1[...]) / cnt
    h = (jnp.dot(mean1, wl1[...], preferred_element_type=jnp.float32)
         + bl1[...]
         + jnp.dot(xr[...], wr1[...], preferred_element_type=jnp.float32))
    mu = jnp.mean(h, axis=0, keepdims=True)
    var = jnp.mean((h - mu) ** 2, axis=0, keepdims=True)
    h = (h - mu) / jnp.sqrt(var + 1e-5) * g1r[...] + be1r[...]
    h = jnp.maximum(h, 0.0)
    oh2[...] = jnp.dot(h, wl2[...], preferred_element_type=jnp.float32)
    ohr[...] = jnp.dot(h, wr2[...], preferred_element_type=jnp.float32)
    ocnt[...] = cnt


def _dense2_body(q0, q1, cnt_r, hr, bl2, g2r, be2r, wc, bc, out):
    h = (q0[...] + q1[...]) / cnt_r[...] + bl2[...] + hr[...]
    mu = jnp.mean(h, axis=0, keepdims=True)
    var = jnp.mean((h - mu) ** 2, axis=0, keepdims=True)
    h = (h - mu) / jnp.sqrt(var + 1e-5) * g2r[...] + be2r[...]
    h = jnp.maximum(h, 0.0)
    out[...] = jnp.dot(h, wc[...], preferred_element_type=jnp.float32) + bc[...]


def kernel(x, edge_index, W_l1, b_l1, W_r1, g1, be1, W_l2, b_l2, W_r2,
           g2, be2, Wc, bc):
    n, d = x.shape
    e = edge_index.shape[1]
    h2 = W_l2.shape[0]
    f32 = jnp.float32

    src1 = edge_index[0]
    dst1 = edge_index[1]

    a0, a1, c0, c1 = _sc_agg(x, src1, dst1, with_cnt=True)

    dense1 = pl.pallas_call(
        _dense1_body,
        out_shape=[
            jax.ShapeDtypeStruct((n, h2), f32),
            jax.ShapeDtypeStruct((n, h2), f32),
            jax.ShapeDtypeStruct((n, 1), f32),
        ],
    )
    h2pre, hr, cnt = dense1(
        a0, a1, c0, c1, x, W_l1.T, b_l1.reshape(1, -1), W_r1.T,
        g1.reshape(1, -1), be1.reshape(1, -1), W_l2.T, W_r2.T)

    q0, q1 = _sc_agg(h2pre, src1, dst1, with_cnt=False)

    wc_pad = jnp.zeros((h2, 8), f32).at[:, :2].set(Wc.T)
    bc_pad = jnp.zeros((1, 8), f32).at[0, :2].set(bc)
    dense2 = pl.pallas_call(
        _dense2_body,
        out_shape=jax.ShapeDtypeStruct((n, 8), f32),
    )
    logits8 = dense2(q0, q1, cnt, hr, b_l2.reshape(1, -1),
                     g2.reshape(1, -1), be2.reshape(1, -1), wc_pad, bc_pad)
    return logits8[:, :2]
